# baseline (speedup 1.0000x reference)
import jax
import jax.numpy as jnp
from jax import lax
from jax.experimental import pallas as pl
from jax.experimental.pallas import tpu as pltpu
from jax.experimental.pallas import tpu_sc as plsc

B = 4096
DIN = 8192
DOUT = 2048
NGROUP = 64
NC, NS, LANES = 2, 16, 16
HIN = DIN // NC
HOUT = DOUT // NC
R = 8
NBUF = 2
CHUNKS = HOUT // LANES

B_SC = 2048
B_TC = B - B_SC
G = B_SC // (NS * R)
RT = 512


def _sc_body(x_hbm, cidx_hbm, out_hbm, idx_v, in0_v, in1_v, out0_v, out1_v,
             in_sem0, in_sem1, out_sem0, out_sem1):
    in_bufs = (in0_v, in1_v)
    out_bufs = (out0_v, out1_v)
    in_sems = (in_sem0, in_sem1)
    out_sems = (out_sem0, out_sem1)
    h = lax.axis_index("c")
    s = lax.axis_index("s")
    row0 = s * (G * R)

    pltpu.sync_copy(cidx_hbm.at[h], idx_v)

    def start_in(block, b):
        pltpu.async_copy(
            x_hbm.at[pl.ds(row0 + block * R, R), pl.ds(h * HIN, HIN)],
            in_bufs[b], in_sems[b])

    def wait_in(b):
        pltpu.make_async_copy(
            x_hbm.at[pl.ds(0, R), pl.ds(0, HIN)], in_bufs[b],
            in_sems[b]).wait()

    def start_out(block, b):
        pltpu.async_copy(
            out_bufs[b],
            out_hbm.at[pl.ds(row0 + block * R, R), pl.ds(h * HOUT, HOUT)],
            out_sems[b])

    def wait_out(b):
        pltpu.make_async_copy(
            out_bufs[b], out_hbm.at[pl.ds(0, R), pl.ds(0, HOUT)],
            out_sems[b]).wait()

    for b in range(NBUF):
        start_in(b, b)

    @pl.loop(0, G, step=NBUF)
    def _outer(g):
        for b in range(NBUF):
            i = g + b
            wait_in(b)

            @pl.when(i >= NBUF)
            def _():
                wait_out(b)

            @plsc.parallel_loop(0, CHUNKS, unroll=4)
            def _inner(o):
                iv = idx_v[pl.ds(o * LANES, LANES)]
                for r in range(R):
                    rv = jnp.full((LANES,), r, jnp.int32)
                    vals = plsc.load_gather(in_bufs[b], [rv, iv])
                    out_bufs[b][r, pl.ds(o * LANES, LANES)] = vals

            start_out(i, b)

            @pl.when(i + NBUF < G)
            def _():
                start_in(i + NBUF, b)

    for b in range(NBUF):
        wait_out(b)


TCG = 4


def _tc_body(s_ref, x_ref, out_ref):
    s = s_ref[...]
    parts = []
    for k in range(TCG):
        parts.append(lax.dot_general(
            x_ref[:, k * 128:(k + 1) * 128], s, (((1,), (0,)), ((), ())),
            precision=lax.Precision.HIGHEST,
            preferred_element_type=jnp.float32))
    out_ref[...] = jnp.concatenate(parts, axis=1)


def kernel(x, weight):
    w32 = weight.astype(jnp.int32)
    cidx = (jnp.arange(NGROUP, dtype=jnp.int32)[:, None] * 128
            + w32[None, :]).reshape(-1)
    cidx = jnp.clip(cidx, 0, DIN - 1)
    halves = []
    for hh in range(NC):
        lo, hi = hh * HIN, (hh + 1) * HIN - 1
        halves.append(jnp.clip(cidx[hh * HOUT:(hh + 1) * HOUT], lo, hi) - lo)
    cidx2 = jnp.stack(halves)

    sel = (jnp.arange(128, dtype=jnp.int32)[:, None]
           == jnp.clip(w32, 0, 127)[None, :]).astype(jnp.float32)

    mesh = plsc.VectorSubcoreMesh(core_axis_name="c", subcore_axis_name="s")
    sc_fn = pl.kernel(
        _sc_body,
        out_type=jax.ShapeDtypeStruct((B_SC, DOUT), jnp.float32),
        mesh=mesh,
        compiler_params=pltpu.CompilerParams(needs_layout_passes=False),
        scratch_types=[
            pltpu.VMEM((HOUT,), jnp.int32),
            pltpu.VMEM((R, HIN), jnp.float32),
            pltpu.VMEM((R, HIN), jnp.float32),
            pltpu.VMEM((R, HOUT), jnp.float32),
            pltpu.VMEM((R, HOUT), jnp.float32),
            pltpu.SemaphoreType.DMA,
            pltpu.SemaphoreType.DMA,
            pltpu.SemaphoreType.DMA,
            pltpu.SemaphoreType.DMA,
        ],
    )
    out_sc = sc_fn(x, cidx2)

    tc_fn = pl.pallas_call(
        _tc_body,
        grid=(B_TC // RT, NGROUP // TCG),
        in_specs=[
            pl.BlockSpec((128, 32), lambda i, g: (0, 0)),
            pl.BlockSpec((RT, 128 * TCG), lambda i, g: (i + B_SC // RT, g)),
        ],
        out_specs=pl.BlockSpec((RT, 32 * TCG), lambda i, g: (i, g)),
        out_shape=jax.ShapeDtypeStruct((B_TC, DOUT), jnp.float32),
    )
    out_tc = tc_fn(sel, x)

    return jnp.concatenate([out_sc, out_tc], axis=0)


# speedup vs baseline: 1.1842x; 1.1842x over previous
import jax
import jax.numpy as jnp
from jax import lax
from jax.experimental import pallas as pl
from jax.experimental.pallas import tpu as pltpu
from jax.experimental.pallas import tpu_sc as plsc

B = 4096
DIN = 8192
DOUT = 2048
NGROUP = 64
NC, NS, LANES = 2, 16, 16
HIN = DIN // NC
HOUT = DOUT // NC
R = 8
NBUF = 2
CHUNKS = HOUT // LANES

B_SC = 2048
B_TC = B - B_SC
G = B_SC // (NS * R)
RT = 512


def _sc_body(x_hbm, cidx_hbm, out_hbm, idx_v, in0_v, in1_v, out0_v, out1_v,
             in_sem0, in_sem1, out_sem0, out_sem1):
    in_bufs = (in0_v, in1_v)
    out_bufs = (out0_v, out1_v)
    in_sems = (in_sem0, in_sem1)
    out_sems = (out_sem0, out_sem1)
    h = lax.axis_index("c")
    s = lax.axis_index("s")
    row0 = s * (G * R)

    pltpu.sync_copy(cidx_hbm.at[h], idx_v)

    def start_in(block, b):
        pltpu.async_copy(
            x_hbm.at[pl.ds(row0 + block * R, R), pl.ds(h * HIN, HIN)],
            in_bufs[b], in_sems[b])

    def wait_in(b):
        pltpu.make_async_copy(
            x_hbm.at[pl.ds(0, R), pl.ds(0, HIN)], in_bufs[b],
            in_sems[b]).wait()

    def start_out(block, b):
        pltpu.async_copy(
            out_bufs[b],
            out_hbm.at[pl.ds(row0 + block * R, R), pl.ds(h * HOUT, HOUT)],
            out_sems[b])

    def wait_out(b):
        pltpu.make_async_copy(
            out_bufs[b], out_hbm.at[pl.ds(0, R), pl.ds(0, HOUT)],
            out_sems[b]).wait()

    for b in range(NBUF):
        start_in(b, b)

    @pl.loop(0, G, step=NBUF)
    def _outer(g):
        for b in range(NBUF):
            i = g + b
            wait_in(b)

            @pl.when(i >= NBUF)
            def _():
                wait_out(b)

            @plsc.parallel_loop(0, CHUNKS, unroll=4)
            def _inner(o):
                iv = idx_v[pl.ds(o * LANES, LANES)]
                for r in range(R):
                    rv = jnp.full((LANES,), r, jnp.int32)
                    vals = plsc.load_gather(in_bufs[b], [rv, iv])
                    out_bufs[b][r, pl.ds(o * LANES, LANES)] = vals

            start_out(i, b)

            @pl.when(i + NBUF < G)
            def _():
                start_in(i + NBUF, b)

    for b in range(NBUF):
        wait_out(b)


TCG = 4


def _tc_body(s_ref, x_ref, out_ref):
    out_ref[...] = lax.dot_general(
        x_ref[...], s_ref[...], (((1,), (0,)), ((), ())),
        preferred_element_type=jnp.float32)


def kernel(x, weight):
    w32 = weight.astype(jnp.int32)
    cidx = (jnp.arange(NGROUP, dtype=jnp.int32)[:, None] * 128
            + w32[None, :]).reshape(-1)
    cidx = jnp.clip(cidx, 0, DIN - 1)
    halves = []
    for hh in range(NC):
        lo, hi = hh * HIN, (hh + 1) * HIN - 1
        halves.append(jnp.clip(cidx[hh * HOUT:(hh + 1) * HOUT], lo, hi) - lo)
    cidx2 = jnp.stack(halves)

    sel = (jnp.arange(128, dtype=jnp.int32)[:, None]
           == jnp.clip(w32, 0, 127)[None, :]).astype(jnp.float32)
    sel_bd = jnp.kron(jnp.eye(TCG, dtype=jnp.float32), sel)

    mesh = plsc.VectorSubcoreMesh(core_axis_name="c", subcore_axis_name="s")
    sc_fn = pl.kernel(
        _sc_body,
        out_type=jax.ShapeDtypeStruct((B_SC, DOUT), jnp.float32),
        mesh=mesh,
        compiler_params=pltpu.CompilerParams(needs_layout_passes=False),
        scratch_types=[
            pltpu.VMEM((HOUT,), jnp.int32),
            pltpu.VMEM((R, HIN), jnp.float32),
            pltpu.VMEM((R, HIN), jnp.float32),
            pltpu.VMEM((R, HOUT), jnp.float32),
            pltpu.VMEM((R, HOUT), jnp.float32),
            pltpu.SemaphoreType.DMA,
            pltpu.SemaphoreType.DMA,
            pltpu.SemaphoreType.DMA,
            pltpu.SemaphoreType.DMA,
        ],
    )
    out_sc = sc_fn(x, cidx2)

    tc_fn = pl.pallas_call(
        _tc_body,
        grid=(B_TC // RT, NGROUP // TCG),
        in_specs=[
            pl.BlockSpec((128 * TCG, 32 * TCG), lambda i, g: (0, 0)),
            pl.BlockSpec((RT, 128 * TCG), lambda i, g: (i + B_SC // RT, g)),
        ],
        out_specs=pl.BlockSpec((RT, 32 * TCG), lambda i, g: (i, g)),
        out_shape=jax.ShapeDtypeStruct((B_TC, DOUT), jnp.float32),
    )
    out_tc = tc_fn(sel_bd, x)

    return jnp.concatenate([out_sc, out_tc], axis=0)


# speedup vs baseline: 1.3334x; 1.1260x over previous
import jax
import jax.numpy as jnp
from jax import lax
from jax.experimental import pallas as pl
from jax.experimental.pallas import tpu as pltpu
from jax.experimental.pallas import tpu_sc as plsc

B = 4096
DIN = 8192
DOUT = 2048
NGROUP = 64
NC, NS, LANES = 2, 16, 16
HIN = DIN // NC
HOUT = DOUT // NC
R = 8
NBUF = 2
CHUNKS = HOUT // LANES

B_SC = 3072
B_TC = B - B_SC
G = B_SC // (NS * R)
RT = 512


def _sc_body(x_hbm, cidx_hbm, out_hbm, idx_v, in0_v, in1_v, out0_v, out1_v,
             in_sem0, in_sem1, out_sem0, out_sem1):
    in_bufs = (in0_v, in1_v)
    out_bufs = (out0_v, out1_v)
    in_sems = (in_sem0, in_sem1)
    out_sems = (out_sem0, out_sem1)
    h = lax.axis_index("c")
    s = lax.axis_index("s")
    row0 = s * (G * R)

    pltpu.sync_copy(cidx_hbm.at[h], idx_v)

    def start_in(block, b):
        pltpu.async_copy(
            x_hbm.at[pl.ds(row0 + block * R, R), pl.ds(h * HIN, HIN)],
            in_bufs[b], in_sems[b])

    def wait_in(b):
        pltpu.make_async_copy(
            x_hbm.at[pl.ds(0, R), pl.ds(0, HIN)], in_bufs[b],
            in_sems[b]).wait()

    def start_out(block, b):
        pltpu.async_copy(
            out_bufs[b],
            out_hbm.at[pl.ds(row0 + block * R, R), pl.ds(h * HOUT, HOUT)],
            out_sems[b])

    def wait_out(b):
        pltpu.make_async_copy(
            out_bufs[b], out_hbm.at[pl.ds(0, R), pl.ds(0, HOUT)],
            out_sems[b]).wait()

    for b in range(NBUF):
        start_in(b, b)

    @pl.loop(0, G, step=NBUF)
    def _outer(g):
        for b in range(NBUF):
            i = g + b
            wait_in(b)

            @pl.when(i >= NBUF)
            def _():
                wait_out(b)

            @plsc.parallel_loop(0, CHUNKS, unroll=4)
            def _inner(o):
                iv = idx_v[pl.ds(o * LANES, LANES)]
                for r in range(R):
                    rv = jnp.full((LANES,), r, jnp.int32)
                    vals = plsc.load_gather(in_bufs[b], [rv, iv])
                    out_bufs[b][r, pl.ds(o * LANES, LANES)] = vals

            start_out(i, b)

            @pl.when(i + NBUF < G)
            def _():
                start_in(i + NBUF, b)

    for b in range(NBUF):
        wait_out(b)


TCG = 8


def _tc_body(s_ref, x_ref, out_ref):
    out_ref[...] = lax.dot_general(
        x_ref[...], s_ref[...], (((1,), (0,)), ((), ())),
        preferred_element_type=jnp.float32)


def kernel(x, weight):
    w32 = weight.astype(jnp.int32)
    cidx = (jnp.arange(NGROUP, dtype=jnp.int32)[:, None] * 128
            + w32[None, :]).reshape(-1)
    cidx = jnp.clip(cidx, 0, DIN - 1)
    halves = []
    for hh in range(NC):
        lo, hi = hh * HIN, (hh + 1) * HIN - 1
        halves.append(jnp.clip(cidx[hh * HOUT:(hh + 1) * HOUT], lo, hi) - lo)
    cidx2 = jnp.stack(halves)

    sel = (jnp.arange(128, dtype=jnp.int32)[:, None]
           == jnp.clip(w32, 0, 127)[None, :]).astype(jnp.float32)
    sel_bd = jnp.kron(jnp.eye(TCG, dtype=jnp.float32), sel)

    mesh = plsc.VectorSubcoreMesh(core_axis_name="c", subcore_axis_name="s")
    sc_fn = pl.kernel(
        _sc_body,
        out_type=jax.ShapeDtypeStruct((B_SC, DOUT), jnp.float32),
        mesh=mesh,
        compiler_params=pltpu.CompilerParams(needs_layout_passes=False),
        scratch_types=[
            pltpu.VMEM((HOUT,), jnp.int32),
            pltpu.VMEM((R, HIN), jnp.float32),
            pltpu.VMEM((R, HIN), jnp.float32),
            pltpu.VMEM((R, HOUT), jnp.float32),
            pltpu.VMEM((R, HOUT), jnp.float32),
            pltpu.SemaphoreType.DMA,
            pltpu.SemaphoreType.DMA,
            pltpu.SemaphoreType.DMA,
            pltpu.SemaphoreType.DMA,
        ],
    )
    out_sc = sc_fn(x, cidx2)

    tc_fn = pl.pallas_call(
        _tc_body,
        grid=(B_TC // RT, NGROUP // TCG),
        in_specs=[
            pl.BlockSpec((128 * TCG, 32 * TCG), lambda i, g: (0, 0)),
            pl.BlockSpec((RT, 128 * TCG), lambda i, g: (i + B_SC // RT, g)),
        ],
        out_specs=pl.BlockSpec((RT, 32 * TCG), lambda i, g: (i, g)),
        out_shape=jax.ShapeDtypeStruct((B_TC, DOUT), jnp.float32),
    )
    out_tc = tc_fn(sel_bd, x)

    return jnp.concatenate([out_sc, out_tc], axis=0)


# speedup vs baseline: 1.5211x; 1.1408x over previous
import jax
import jax.numpy as jnp
from jax import lax
from jax.experimental import pallas as pl
from jax.experimental.pallas import tpu as pltpu
from jax.experimental.pallas import tpu_sc as plsc

B = 4096
DIN = 8192
DOUT = 2048
NGROUP = 64
NC, NS, LANES = 2, 16, 16
HIN = DIN // NC
HOUT = DOUT // NC
R = 8
NBUF = 2
CHUNKS = HOUT // LANES

B_SC = 2560
B_TC = B - B_SC
G = B_SC // (NS * R)
RT = 512


def _sc_body(x_hbm, cidx_hbm, out_hbm, idx_v, in0_v, in1_v, out0_v, out1_v,
             in_sem0, in_sem1, out_sem0, out_sem1):
    in_bufs = (in0_v, in1_v)
    out_bufs = (out0_v, out1_v)
    in_sems = (in_sem0, in_sem1)
    out_sems = (out_sem0, out_sem1)
    h = lax.axis_index("c")
    s = lax.axis_index("s")
    row0 = s * (G * R)

    pltpu.sync_copy(cidx_hbm.at[h], idx_v)

    def start_in(block, b):
        pltpu.async_copy(
            x_hbm.at[pl.ds(row0 + block * R, R), pl.ds(h * HIN, HIN)],
            in_bufs[b], in_sems[b])

    def wait_in(b):
        pltpu.make_async_copy(
            x_hbm.at[pl.ds(0, R), pl.ds(0, HIN)], in_bufs[b],
            in_sems[b]).wait()

    def start_out(block, b):
        pltpu.async_copy(
            out_bufs[b],
            out_hbm.at[pl.ds(row0 + block * R, R), pl.ds(h * HOUT, HOUT)],
            out_sems[b])

    def wait_out(b):
        pltpu.make_async_copy(
            out_bufs[b], out_hbm.at[pl.ds(0, R), pl.ds(0, HOUT)],
            out_sems[b]).wait()

    for b in range(NBUF):
        start_in(b, b)

    @pl.loop(0, G, step=NBUF)
    def _outer(g):
        for b in range(NBUF):
            i = g + b
            wait_in(b)

            @pl.when(i >= NBUF)
            def _():
                wait_out(b)

            @plsc.parallel_loop(0, CHUNKS, unroll=4)
            def _inner(o):
                iv = idx_v[pl.ds(o * LANES, LANES)]
                for r in range(R):
                    rv = jnp.full((LANES,), r, jnp.int32)
                    vals = plsc.load_gather(in_bufs[b], [rv, iv])
                    out_bufs[b][r, pl.ds(o * LANES, LANES)] = vals

            start_out(i, b)

            @pl.when(i + NBUF < G)
            def _():
                start_in(i + NBUF, b)

    for b in range(NBUF):
        wait_out(b)


TCG = 8


def _tc_body(s_ref, x_ref, out_ref):
    out_ref[...] = lax.dot_general(
        x_ref[...], s_ref[...], (((1,), (0,)), ((), ())),
        preferred_element_type=jnp.float32)


def kernel(x, weight):
    w32 = weight.astype(jnp.int32)
    cidx = (jnp.arange(NGROUP, dtype=jnp.int32)[:, None] * 128
            + w32[None, :]).reshape(-1)
    cidx = jnp.clip(cidx, 0, DIN - 1)
    halves = []
    for hh in range(NC):
        lo, hi = hh * HIN, (hh + 1) * HIN - 1
        halves.append(jnp.clip(cidx[hh * HOUT:(hh + 1) * HOUT], lo, hi) - lo)
    cidx2 = jnp.stack(halves)

    sel = (jnp.arange(128, dtype=jnp.int32)[:, None]
           == jnp.clip(w32, 0, 127)[None, :]).astype(jnp.float32)
    sel_bd = jnp.kron(jnp.eye(TCG, dtype=jnp.float32), sel)

    mesh = plsc.VectorSubcoreMesh(core_axis_name="c", subcore_axis_name="s")
    sc_fn = pl.kernel(
        _sc_body,
        out_type=jax.ShapeDtypeStruct((B, DOUT), jnp.float32),
        mesh=mesh,
        compiler_params=pltpu.CompilerParams(needs_layout_passes=False),
        scratch_types=[
            pltpu.VMEM((HOUT,), jnp.int32),
            pltpu.VMEM((R, HIN), jnp.float32),
            pltpu.VMEM((R, HIN), jnp.float32),
            pltpu.VMEM((R, HOUT), jnp.float32),
            pltpu.VMEM((R, HOUT), jnp.float32),
            pltpu.SemaphoreType.DMA,
            pltpu.SemaphoreType.DMA,
            pltpu.SemaphoreType.DMA,
            pltpu.SemaphoreType.DMA,
        ],
    )
    out_sc = sc_fn(x, cidx2)

    tc_fn = pl.pallas_call(
        _tc_body,
        grid=(B_TC // RT, NGROUP // TCG),
        in_specs=[
            pl.BlockSpec((128 * TCG, 32 * TCG), lambda i, g: (0, 0)),
            pl.BlockSpec((RT, 128 * TCG), lambda i, g: (i + B_SC // RT, g)),
        ],
        out_specs=pl.BlockSpec((RT, 32 * TCG), lambda i, g: (i, g)),
        out_shape=jax.ShapeDtypeStruct((B_TC, DOUT), jnp.float32),
    )
    out_tc = tc_fn(sel_bd, x)

    return lax.dynamic_update_slice(out_sc, out_tc, (B_SC, 0))


# speedup vs baseline: 1.7014x; 1.1185x over previous
import jax
import jax.numpy as jnp
from jax import lax
from jax.experimental import pallas as pl
from jax.experimental.pallas import tpu as pltpu
from jax.experimental.pallas import tpu_sc as plsc

B = 4096
DIN = 8192
DOUT = 2048
NC, NS, LANES = 2, 16, 16
HIN = DIN // NC
HOUT = DOUT // NC
R = 8
NBUF = 2
G = B // (NS * R)
CHUNKS = HOUT // LANES
HW = HIN // 2


def _body(x_hbm, cidx_hbm, out_hbm, idx_v, in0_v, in1_v, out0_v, out1_v,
          ina_sem0, ina_sem1, inb_sem0, inb_sem1, out_sem0, out_sem1):
    in_bufs = (in0_v, in1_v)
    out_bufs = (out0_v, out1_v)
    ina_sems = (ina_sem0, ina_sem1)
    inb_sems = (inb_sem0, inb_sem1)
    out_sems = (out_sem0, out_sem1)
    h = lax.axis_index("c")
    s = lax.axis_index("s")
    row0 = s * (G * R)

    pltpu.sync_copy(cidx_hbm.at[h], idx_v)

    def start_in(block, b):
        rs = pl.ds(row0 + block * R, R)
        pltpu.async_copy(x_hbm.at[rs, pl.ds(h * HIN, HW)],
                         in_bufs[b].at[:, pl.ds(0, HW)], ina_sems[b])
        pltpu.async_copy(x_hbm.at[rs, pl.ds(h * HIN + HW, HW)],
                         in_bufs[b].at[:, pl.ds(HW, HW)], inb_sems[b])

    def wait_in(b):
        pltpu.make_async_copy(x_hbm.at[pl.ds(0, R), pl.ds(0, HW)],
                              in_bufs[b].at[:, pl.ds(0, HW)],
                              ina_sems[b]).wait()
        pltpu.make_async_copy(x_hbm.at[pl.ds(0, R), pl.ds(0, HW)],
                              in_bufs[b].at[:, pl.ds(0, HW)],
                              inb_sems[b]).wait()

    def start_out(block, b):
        pltpu.async_copy(
            out_bufs[b],
            out_hbm.at[pl.ds(row0 + block * R, R), pl.ds(h * HOUT, HOUT)],
            out_sems[b])

    def wait_out(b):
        pltpu.make_async_copy(
            out_bufs[b], out_hbm.at[pl.ds(0, R), pl.ds(0, HOUT)],
            out_sems[b]).wait()

    for b in range(NBUF):
        start_in(b, b)

    @pl.loop(0, G, step=NBUF)
    def _outer(g):
        for b in range(NBUF):
            i = g + b
            wait_in(b)

            @pl.when(i >= NBUF)
            def _():
                wait_out(b)

            @plsc.parallel_loop(0, CHUNKS, unroll=4)
            def _inner(o):
                iv = idx_v[pl.ds(o * LANES, LANES)]
                for r in range(R):
                    rv = jnp.full((LANES,), r, jnp.int32)
                    vals = plsc.load_gather(in_bufs[b], [rv, iv])
                    out_bufs[b][r, pl.ds(o * LANES, LANES)] = vals

            start_out(i, b)

            @pl.when(i + NBUF < G)
            def _():
                start_in(i + NBUF, b)

    for b in range(NBUF):
        wait_out(b)


def kernel(x, weight):
    cidx = (jnp.arange(DIN // 128, dtype=jnp.int32)[:, None] * 128
            + weight.astype(jnp.int32)[None, :]).reshape(-1)
    cidx = jnp.clip(cidx, 0, DIN - 1)
    halves = []
    for hh in range(NC):
        lo, hi = hh * HIN, (hh + 1) * HIN - 1
        halves.append(jnp.clip(cidx[hh * HOUT:(hh + 1) * HOUT], lo, hi) - lo)
    cidx2 = jnp.stack(halves)

    mesh = plsc.VectorSubcoreMesh(core_axis_name="c", subcore_axis_name="s")
    f = pl.kernel(
        _body,
        out_type=jax.ShapeDtypeStruct((B, DOUT), jnp.float32),
        mesh=mesh,
        compiler_params=pltpu.CompilerParams(needs_layout_passes=False),
        scratch_types=[
            pltpu.VMEM((HOUT,), jnp.int32),
            pltpu.VMEM((R, HIN), jnp.float32),
            pltpu.VMEM((R, HIN), jnp.float32),
            pltpu.VMEM((R, HOUT), jnp.float32),
            pltpu.VMEM((R, HOUT), jnp.float32),
            pltpu.SemaphoreType.DMA,
            pltpu.SemaphoreType.DMA,
            pltpu.SemaphoreType.DMA,
            pltpu.SemaphoreType.DMA,
            pltpu.SemaphoreType.DMA,
            pltpu.SemaphoreType.DMA,
        ],
    )
    return f(x, cidx2)


# speedup vs baseline: 1.7125x; 1.0065x over previous
import jax
import jax.numpy as jnp
from jax import lax
from jax.experimental import pallas as pl
from jax.experimental.pallas import tpu as pltpu
from jax.experimental.pallas import tpu_sc as plsc

B = 4096
DIN = 8192
DOUT = 2048
NC, NS, LANES = 2, 16, 16
HIN = DIN // NC
HOUT = DOUT // NC
R = 8
NBUF = 2
G = B // (NS * R)
CHUNKS = HOUT // LANES
NSTR = 4
HW = HIN // NSTR


def _body(x_hbm, cidx_hbm, out_hbm, idx_v, in0_v, in1_v, out0_v, out1_v,
          s00, s01, s02, s03, s10, s11, s12, s13, out_sem0, out_sem1):
    in_bufs = (in0_v, in1_v)
    out_bufs = (out0_v, out1_v)
    in_sems = ((s00, s01, s02, s03), (s10, s11, s12, s13))
    out_sems = (out_sem0, out_sem1)
    h = lax.axis_index("c")
    s = lax.axis_index("s")
    row0 = s * (G * R)

    pltpu.sync_copy(cidx_hbm.at[h], idx_v)

    def start_in(block, b):
        rs = pl.ds(row0 + block * R, R)
        for k in range(NSTR):
            pltpu.async_copy(x_hbm.at[rs, pl.ds(h * HIN + k * HW, HW)],
                             in_bufs[b].at[:, pl.ds(k * HW, HW)],
                             in_sems[b][k])

    def wait_in(b):
        for k in range(NSTR):
            pltpu.make_async_copy(x_hbm.at[pl.ds(0, R), pl.ds(0, HW)],
                                  in_bufs[b].at[:, pl.ds(0, HW)],
                                  in_sems[b][k]).wait()

    def start_out(block, b):
        pltpu.async_copy(
            out_bufs[b],
            out_hbm.at[pl.ds(row0 + block * R, R), pl.ds(h * HOUT, HOUT)],
            out_sems[b])

    def wait_out(b):
        pltpu.make_async_copy(
            out_bufs[b], out_hbm.at[pl.ds(0, R), pl.ds(0, HOUT)],
            out_sems[b]).wait()

    for b in range(NBUF):
        start_in(b, b)

    @pl.loop(0, G, step=NBUF)
    def _outer(g):
        for b in range(NBUF):
            i = g + b
            wait_in(b)

            @pl.when(i >= NBUF)
            def _():
                wait_out(b)

            @plsc.parallel_loop(0, CHUNKS, unroll=4)
            def _inner(o):
                iv = idx_v[pl.ds(o * LANES, LANES)]
                for r in range(R):
                    rv = jnp.full((LANES,), r, jnp.int32)
                    vals = plsc.load_gather(in_bufs[b], [rv, iv])
                    out_bufs[b][r, pl.ds(o * LANES, LANES)] = vals

            start_out(i, b)

            @pl.when(i + NBUF < G)
            def _():
                start_in(i + NBUF, b)

    for b in range(NBUF):
        wait_out(b)


def kernel(x, weight):
    cidx = (jnp.arange(DIN // 128, dtype=jnp.int32)[:, None] * 128
            + weight.astype(jnp.int32)[None, :]).reshape(-1)
    cidx = jnp.clip(cidx, 0, DIN - 1)
    halves = []
    for hh in range(NC):
        lo, hi = hh * HIN, (hh + 1) * HIN - 1
        halves.append(jnp.clip(cidx[hh * HOUT:(hh + 1) * HOUT], lo, hi) - lo)
    cidx2 = jnp.stack(halves)

    mesh = plsc.VectorSubcoreMesh(core_axis_name="c", subcore_axis_name="s")
    f = pl.kernel(
        _body,
        out_type=jax.ShapeDtypeStruct((B, DOUT), jnp.float32),
        mesh=mesh,
        compiler_params=pltpu.CompilerParams(needs_layout_passes=False),
        scratch_types=[
            pltpu.VMEM((HOUT,), jnp.int32),
            pltpu.VMEM((R, HIN), jnp.float32),
            pltpu.VMEM((R, HIN), jnp.float32),
            pltpu.VMEM((R, HOUT), jnp.float32),
            pltpu.VMEM((R, HOUT), jnp.float32),
            pltpu.SemaphoreType.DMA,
            pltpu.SemaphoreType.DMA,
            pltpu.SemaphoreType.DMA,
            pltpu.SemaphoreType.DMA,
            pltpu.SemaphoreType.DMA,
            pltpu.SemaphoreType.DMA,
            pltpu.SemaphoreType.DMA,
            pltpu.SemaphoreType.DMA,
            pltpu.SemaphoreType.DMA,
            pltpu.SemaphoreType.DMA,
        ],
    )
    return f(x, cidx2)
